# Initial kernel scaffold; baseline (speedup 1.0000x reference)
import jax
import jax.numpy as jnp
from jax.experimental import pallas as pl


def kernel(input_labels, out_labels, in_embed, out_embed, edge_w, u_noise, v_noise, num_sampled):
    raise NotImplementedError("write your pallas kernel here")


# speedup vs baseline: 1.0938x; 1.0938x over previous
import functools

import jax
import jax.numpy as jnp
from jax import lax
from jax.experimental import pallas as pl
from jax.experimental.pallas import tpu as pltpu
from jax.experimental.pallas import tpu_sc as plsc

L = 16
NC = 2
NSC = 16
NT = NC * NSC
SUB = 16
WEIGHT_DECAY = 0.001


def _sc_dot_kernel(B, W, NS, D, NSUB):
    mesh = plsc.VectorSubcoreMesh(core_axis_name="c", subcore_axis_name="s")
    f32 = jnp.float32
    KD = D // L

    @functools.partial(
        pl.kernel,
        mesh=mesh,
        compiler_params=pltpu.CompilerParams(
            needs_layout_passes=False, use_tc_tiling_on_sc=False),
        out_type=[
            jax.ShapeDtypeStruct((NT, NSUB * SUB), f32),
            jax.ShapeDtypeStruct((NT, NSUB * SUB), f32),
            jax.ShapeDtypeStruct((NT, NSUB * NS * SUB), f32),
            jax.ShapeDtypeStruct((NT, NSUB * NS * SUB), f32),
            jax.ShapeDtypeStruct((NT, NSUB * NS * SUB), f32),
            jax.ShapeDtypeStruct((NT, NSUB * NS * SUB), f32),
            jax.ShapeDtypeStruct((NT, L), f32),
        ],
        scratch_types=[
            pltpu.VMEM((NSUB, SUB), jnp.int32),
            pltpu.VMEM((NSUB, SUB), jnp.int32),
            pltpu.VMEM((NSUB, NS * SUB), jnp.int32),
            pltpu.VMEM((NSUB, NS * SUB), jnp.int32),
            pltpu.VMEM((D,), f32),
            pltpu.VMEM((SUB, D), f32),
            pltpu.VMEM((SUB, D), f32),
            pltpu.VMEM((SUB, D), f32),
            pltpu.VMEM((SUB, D), f32),
            pltpu.VMEM((NS * SUB, D), f32),
            pltpu.VMEM((NS * SUB, D), f32),
            pltpu.VMEM((NS * SUB, D), f32),
            pltpu.VMEM((NS * SUB, D), f32),
            pltpu.VMEM((NSUB * SUB,), f32),
            pltpu.VMEM((NSUB * SUB,), f32),
            pltpu.VMEM((NSUB * NS * SUB,), f32),
            pltpu.VMEM((NSUB * NS * SUB,), f32),
            pltpu.VMEM((NSUB * NS * SUB,), f32),
            pltpu.VMEM((NSUB * NS * SUB,), f32),
            pltpu.VMEM((L,), f32),
            pltpu.SemaphoreType.DMA,
        ],
    )
    def k(in_hbm, out_hbm, w_hbm, inp_hbm, oid_hbm, un_hbm, vn_hbm,
          tin_o, tout_o, q1_o, q2_o, q3_o, q4_o, regp_o,
          inp_v, oid_v, un_v, vn_v, w_v,
          a_v, b_v, c_v, d_v, un1_v, un2_v, vn1_v, vn2_v,
          tin_b, tout_b, q1_b, q2_b, q3_b, q4_b, racc_b, sem):
        t = lax.axis_index("s") * NC + lax.axis_index("c")
        pltpu.sync_copy(inp_hbm.at[t], inp_v)
        pltpu.sync_copy(oid_hbm.at[t], oid_v)
        pltpu.sync_copy(un_hbm.at[t], un_v)
        pltpu.sync_copy(vn_hbm.at[t], vn_v)
        pltpu.sync_copy(w_hbm, w_v)
        racc_b[...] = jnp.zeros((L,), f32)

        w = [w_v[pl.ds(kk * L, L)] for kk in range(KD)]
        iota = lax.iota(jnp.int32, L)
        masks = [iota == p for p in range(SUB)]

        def body(s, carry):
            cps = [
                pltpu.async_copy(in_hbm.at[inp_v.at[s]], a_v, sem),
                pltpu.async_copy(out_hbm.at[inp_v.at[s]], b_v, sem),
                pltpu.async_copy(in_hbm.at[oid_v.at[s]], c_v, sem),
                pltpu.async_copy(out_hbm.at[oid_v.at[s]], d_v, sem),
                pltpu.async_copy(in_hbm.at[un_v.at[s]], un1_v, sem),
                pltpu.async_copy(out_hbm.at[un_v.at[s]], un2_v, sem),
                pltpu.async_copy(in_hbm.at[vn_v.at[s]], vn1_v, sem),
                pltpu.async_copy(out_hbm.at[vn_v.at[s]], vn2_v, sem),
            ]
            for cp in cps:
                cp.wait()

            racc = jnp.zeros((L,), f32)
            zero = jnp.zeros((L,), f32)
            tin_res = zero
            tout_res = zero
            q_res = [[zero] * NS for _ in range(4)]
            for p in range(SUB):
                a = [a_v[p, pl.ds(kk * L, L)] for kk in range(KD)]
                b = [b_v[p, pl.ds(kk * L, L)] for kk in range(KD)]
                c = [c_v[p, pl.ds(kk * L, L)] for kk in range(KD)]
                d = [d_v[p, pl.ds(kk * L, L)] for kk in range(KD)]
                aw = [a[kk] * w[kk] for kk in range(KD)]
                bw = [b[kk] * w[kk] for kk in range(KD)]
                cw = [c[kk] * w[kk] for kk in range(KD)]
                dw = [d[kk] * w[kk] for kk in range(KD)]
                tinv = aw[0] * c[0]
                toutv = bw[0] * d[0]
                sq = a[0] * a[0] + b[0] * b[0] + c[0] * c[0] + d[0] * d[0]
                for kk in range(1, KD):
                    tinv = tinv + aw[kk] * c[kk]
                    toutv = toutv + bw[kk] * d[kk]
                    sq = sq + a[kk] * a[kk] + b[kk] * b[kk] \
                        + c[kk] * c[kk] + d[kk] * d[kk]
                racc = racc + sq
                tin_res = jnp.where(masks[p], jnp.sum(tinv), tin_res)
                tout_res = jnp.where(masks[p], jnp.sum(toutv), tout_res)
                for j in range(NS):
                    r = p * NS + j
                    u1 = [un1_v[r, pl.ds(kk * L, L)] for kk in range(KD)]
                    u2 = [un2_v[r, pl.ds(kk * L, L)] for kk in range(KD)]
                    v1 = [vn1_v[r, pl.ds(kk * L, L)] for kk in range(KD)]
                    v2 = [vn2_v[r, pl.ds(kk * L, L)] for kk in range(KD)]
                    q1v = u1[0] * cw[0]
                    q2v = u2[0] * dw[0]
                    q3v = v1[0] * aw[0]
                    q4v = v2[0] * bw[0]
                    nsq = u1[0] * u1[0] + u2[0] * u2[0] \
                        + v1[0] * v1[0] + v2[0] * v2[0]
                    for kk in range(1, KD):
                        q1v = q1v + u1[kk] * cw[kk]
                        q2v = q2v + u2[kk] * dw[kk]
                        q3v = q3v + v1[kk] * aw[kk]
                        q4v = q4v + v2[kk] * bw[kk]
                        nsq = nsq + u1[kk] * u1[kk] + u2[kk] * u2[kk] \
                            + v1[kk] * v1[kk] + v2[kk] * v2[kk]
                    racc = racc + nsq
                    q_res[0][j] = jnp.where(masks[p], jnp.sum(q1v), q_res[0][j])
                    q_res[1][j] = jnp.where(masks[p], jnp.sum(q2v), q_res[1][j])
                    q_res[2][j] = jnp.where(masks[p], jnp.sum(q3v), q_res[2][j])
                    q_res[3][j] = jnp.where(masks[p], jnp.sum(q4v), q_res[3][j])
            toff = pl.multiple_of(s * SUB, SUB)
            tin_b[pl.ds(toff, SUB)] = tin_res
            tout_b[pl.ds(toff, SUB)] = tout_res
            for j in range(NS):
                qoff = pl.multiple_of((s * NS + j) * SUB, SUB)
                q1_b[pl.ds(qoff, SUB)] = q_res[0][j]
                q2_b[pl.ds(qoff, SUB)] = q_res[1][j]
                q3_b[pl.ds(qoff, SUB)] = q_res[2][j]
                q4_b[pl.ds(qoff, SUB)] = q_res[3][j]
            racc_b[...] = racc_b[...] + racc
            return carry

        lax.fori_loop(0, NSUB, body, 0)
        pltpu.sync_copy(tin_b, tin_o.at[t])
        pltpu.sync_copy(tout_b, tout_o.at[t])
        pltpu.sync_copy(q1_b, q1_o.at[t])
        pltpu.sync_copy(q2_b, q2_o.at[t])
        pltpu.sync_copy(q3_b, q3_o.at[t])
        pltpu.sync_copy(q4_b, q4_o.at[t])
        pltpu.sync_copy(racc_b, regp_o.at[t])

    return k


def _tc_reduce(tin, tout, q1, q2, q3, q4, regp, batch_size):
    def body(tin_ref, tout_ref, q1_ref, q2_ref, q3_ref, q4_ref, regp_ref, o_ref):
        def ls(x):
            return jnp.log(jax.nn.sigmoid(jnp.clip(x, -6.0, 6.0)))

        loss = (jnp.sum(ls(tin_ref[...])) + jnp.sum(ls(tout_ref[...]))
                + 0.5 * (jnp.sum(ls(-q1_ref[...])) + jnp.sum(ls(-q2_ref[...]))
                         + jnp.sum(ls(-q3_ref[...])) + jnp.sum(ls(-q4_ref[...]))))
        total = loss - WEIGHT_DECAY * 0.5 * jnp.sum(regp_ref[...])
        o_ref[0, 0] = -total / batch_size

    return pl.pallas_call(
        body,
        out_shape=jax.ShapeDtypeStruct((1, 1), jnp.float32),
        out_specs=pl.BlockSpec(memory_space=pltpu.SMEM),
    )(tin, tout, q1, q2, q3, q4, regp)


def kernel(input_labels, out_labels, in_embed, out_embed, edge_w, u_noise,
           v_noise, num_sampled):
    B, W1 = out_labels.shape
    W = W1 - 1
    D = in_embed.shape[1]
    NS = u_noise.shape[1]
    pairs = B * W
    ppt = pairs // NT
    nsub = ppt // SUB

    ids = input_labels[:, 1].astype(jnp.int32)
    inp_idx = jnp.tile(ids, W).reshape(NT, nsub, SUB)
    out_idx = out_labels[:, 1:].reshape(-1).astype(jnp.int32).reshape(NT, nsub, SUB)
    un_idx = u_noise.astype(jnp.int32).reshape(NT, nsub, NS * SUB)
    vn_idx = v_noise.astype(jnp.int32).reshape(NT, nsub, NS * SUB)

    tin, tout, q1, q2, q3, q4, regp = _sc_dot_kernel(B, W, NS, D, nsub)(
        in_embed, out_embed, edge_w.astype(jnp.float32), inp_idx, out_idx,
        un_idx, vn_idx)

    res = _tc_reduce(
        tin.reshape(pairs // 128, 128), tout.reshape(pairs // 128, 128),
        q1.reshape(pairs * NS // 128, 128), q2.reshape(pairs * NS // 128, 128),
        q3.reshape(pairs * NS // 128, 128), q4.reshape(pairs * NS // 128, 128),
        regp.reshape((NT * L) // 128, 128), float(B))
    return res[0, 0]


# speedup vs baseline: 1.1262x; 1.0296x over previous
import functools

import jax
import jax.numpy as jnp
from jax import lax
from jax.experimental import pallas as pl
from jax.experimental.pallas import tpu as pltpu
from jax.experimental.pallas import tpu_sc as plsc

L = 16
NC = 2
NSC = 16
NT = NC * NSC
SUB = 16
WEIGHT_DECAY = 0.001


def _sc_dot_kernel(B, W, NS, D, NSUB):
    mesh = plsc.VectorSubcoreMesh(core_axis_name="c", subcore_axis_name="s")
    f32 = jnp.float32
    KD = D // L
    NA = 2 * SUB + NS * SUB
    NB = NS * SUB

    @functools.partial(
        pl.kernel,
        mesh=mesh,
        compiler_params=pltpu.CompilerParams(
            needs_layout_passes=False, use_tc_tiling_on_sc=False),
        out_type=[
            jax.ShapeDtypeStruct((NT, NSUB * SUB), f32),
            jax.ShapeDtypeStruct((NT, NSUB * SUB), f32),
            jax.ShapeDtypeStruct((NT, NSUB * NS * SUB), f32),
            jax.ShapeDtypeStruct((NT, NSUB * NS * SUB), f32),
            jax.ShapeDtypeStruct((NT, NSUB * NS * SUB), f32),
            jax.ShapeDtypeStruct((NT, NSUB * NS * SUB), f32),
            jax.ShapeDtypeStruct((NT, L), f32),
        ],
        scratch_types=[
            pltpu.VMEM((NSUB, NA), jnp.int32),
            pltpu.VMEM((NSUB, NB), jnp.int32),
            pltpu.VMEM((D,), f32),
            pltpu.VMEM((NA, D), f32),
            pltpu.VMEM((NA, D), f32),
            pltpu.VMEM((NB, D), f32),
            pltpu.VMEM((NB, D), f32),
            pltpu.VMEM((NA, D), f32),
            pltpu.VMEM((NA, D), f32),
            pltpu.VMEM((NB, D), f32),
            pltpu.VMEM((NB, D), f32),
            pltpu.VMEM((NSUB * SUB,), f32),
            pltpu.VMEM((NSUB * SUB,), f32),
            pltpu.VMEM((NSUB * NS * SUB,), f32),
            pltpu.VMEM((NSUB * NS * SUB,), f32),
            pltpu.VMEM((NSUB * NS * SUB,), f32),
            pltpu.VMEM((NSUB * NS * SUB,), f32),
            pltpu.VMEM((L,), f32),
            pltpu.SemaphoreType.DMA,
            pltpu.SemaphoreType.DMA,
        ],
    )
    def k(in_hbm, out_hbm, w_hbm, idxa_hbm, idxb_hbm,
          tin_o, tout_o, q1_o, q2_o, q3_o, q4_o, regp_o,
          idxa_v, idxb_v, w_v,
          ai0, ao0, bi0, bo0, ai1, ao1, bi1, bo1,
          tin_b, tout_b, q1_b, q2_b, q3_b, q4_b, racc_b, sem0, sem1):
        t = lax.axis_index("s") * NC + lax.axis_index("c")
        pltpu.sync_copy(idxa_hbm.at[t], idxa_v)
        pltpu.sync_copy(idxb_hbm.at[t], idxb_v)
        pltpu.sync_copy(w_hbm, w_v)
        racc_b[...] = jnp.zeros((L,), f32)

        w = [w_v[pl.ds(kk * L, L)] for kk in range(KD)]
        iota = lax.iota(jnp.int32, L)
        masks = [iota == p for p in range(SUB)]

        def issue(s, ai, ao, bi, bo, sem):
            pltpu.async_copy(in_hbm.at[idxa_v.at[s]], ai, sem)
            pltpu.async_copy(out_hbm.at[idxa_v.at[s]], ao, sem)
            pltpu.async_copy(in_hbm.at[idxb_v.at[s]], bi, sem)
            pltpu.async_copy(out_hbm.at[idxb_v.at[s]], bo, sem)

        def drain(s, ai, ao, bi, bo, sem):
            pltpu.make_async_copy(in_hbm.at[idxa_v.at[s]], ai, sem).wait()
            pltpu.make_async_copy(out_hbm.at[idxa_v.at[s]], ao, sem).wait()
            pltpu.make_async_copy(in_hbm.at[idxb_v.at[s]], bi, sem).wait()
            pltpu.make_async_copy(out_hbm.at[idxb_v.at[s]], bo, sem).wait()

        def compute(s, ai, ao, bi, bo):
            racc = jnp.zeros((L,), f32)
            zero = jnp.zeros((L,), f32)
            tin_res = zero
            tout_res = zero
            q_res = [[zero] * NS for _ in range(4)]
            for p in range(SUB):
                a = [ai[p, pl.ds(kk * L, L)] for kk in range(KD)]
                c = [ai[SUB + p, pl.ds(kk * L, L)] for kk in range(KD)]
                b = [ao[p, pl.ds(kk * L, L)] for kk in range(KD)]
                d = [ao[SUB + p, pl.ds(kk * L, L)] for kk in range(KD)]
                aw = [a[kk] * w[kk] for kk in range(KD)]
                bw = [b[kk] * w[kk] for kk in range(KD)]
                cw = [c[kk] * w[kk] for kk in range(KD)]
                dw = [d[kk] * w[kk] for kk in range(KD)]
                tinv = aw[0] * c[0]
                toutv = bw[0] * d[0]
                sq = a[0] * a[0] + b[0] * b[0] + c[0] * c[0] + d[0] * d[0]
                for kk in range(1, KD):
                    tinv = tinv + aw[kk] * c[kk]
                    toutv = toutv + bw[kk] * d[kk]
                    sq = sq + a[kk] * a[kk] + b[kk] * b[kk] \
                        + c[kk] * c[kk] + d[kk] * d[kk]
                racc = racc + sq
                tin_res = jnp.where(masks[p], jnp.sum(tinv), tin_res)
                tout_res = jnp.where(masks[p], jnp.sum(toutv), tout_res)
                for j in range(NS):
                    r = p * NS + j
                    u1 = [ai[2 * SUB + r, pl.ds(kk * L, L)] for kk in range(KD)]
                    u2 = [ao[2 * SUB + r, pl.ds(kk * L, L)] for kk in range(KD)]
                    v1 = [bi[r, pl.ds(kk * L, L)] for kk in range(KD)]
                    v2 = [bo[r, pl.ds(kk * L, L)] for kk in range(KD)]
                    q1v = u1[0] * cw[0]
                    q2v = u2[0] * dw[0]
                    q3v = v1[0] * aw[0]
                    q4v = v2[0] * bw[0]
                    nsq = u1[0] * u1[0] + u2[0] * u2[0] \
                        + v1[0] * v1[0] + v2[0] * v2[0]
                    for kk in range(1, KD):
                        q1v = q1v + u1[kk] * cw[kk]
                        q2v = q2v + u2[kk] * dw[kk]
                        q3v = q3v + v1[kk] * aw[kk]
                        q4v = q4v + v2[kk] * bw[kk]
                        nsq = nsq + u1[kk] * u1[kk] + u2[kk] * u2[kk] \
                            + v1[kk] * v1[kk] + v2[kk] * v2[kk]
                    racc = racc + nsq
                    q_res[0][j] = jnp.where(masks[p], jnp.sum(q1v), q_res[0][j])
                    q_res[1][j] = jnp.where(masks[p], jnp.sum(q2v), q_res[1][j])
                    q_res[2][j] = jnp.where(masks[p], jnp.sum(q3v), q_res[2][j])
                    q_res[3][j] = jnp.where(masks[p], jnp.sum(q4v), q_res[3][j])
            toff = pl.multiple_of(s * SUB, SUB)
            tin_b[pl.ds(toff, SUB)] = tin_res
            tout_b[pl.ds(toff, SUB)] = tout_res
            for j in range(NS):
                qoff = pl.multiple_of((s * NS + j) * SUB, SUB)
                q1_b[pl.ds(qoff, SUB)] = q_res[0][j]
                q2_b[pl.ds(qoff, SUB)] = q_res[1][j]
                q3_b[pl.ds(qoff, SUB)] = q_res[2][j]
                q4_b[pl.ds(qoff, SUB)] = q_res[3][j]
            racc_b[...] = racc_b[...] + racc

        issue(0, ai0, ao0, bi0, bo0, sem0)

        def body(g, carry):
            s0 = g * 2
            s1 = g * 2 + 1
            issue(s1, ai1, ao1, bi1, bo1, sem1)
            drain(s0, ai0, ao0, bi0, bo0, sem0)
            compute(s0, ai0, ao0, bi0, bo0)
            s2 = jnp.minimum(s1 + 1, NSUB - 1)
            issue(s2, ai0, ao0, bi0, bo0, sem0)
            drain(s1, ai1, ao1, bi1, bo1, sem1)
            compute(s1, ai1, ao1, bi1, bo1)
            return carry

        lax.fori_loop(0, NSUB // 2, body, 0)
        drain(NSUB - 1, ai0, ao0, bi0, bo0, sem0)

        pltpu.sync_copy(tin_b, tin_o.at[t])
        pltpu.sync_copy(tout_b, tout_o.at[t])
        pltpu.sync_copy(q1_b, q1_o.at[t])
        pltpu.sync_copy(q2_b, q2_o.at[t])
        pltpu.sync_copy(q3_b, q3_o.at[t])
        pltpu.sync_copy(q4_b, q4_o.at[t])
        pltpu.sync_copy(racc_b, regp_o.at[t])

    return k


def _tc_reduce(tin, tout, q1, q2, q3, q4, regp, batch_size):
    def body(tin_ref, tout_ref, q1_ref, q2_ref, q3_ref, q4_ref, regp_ref, o_ref):
        def ls(x):
            return jnp.log(jax.nn.sigmoid(jnp.clip(x, -6.0, 6.0)))

        loss = (jnp.sum(ls(tin_ref[...])) + jnp.sum(ls(tout_ref[...]))
                + 0.5 * (jnp.sum(ls(-q1_ref[...])) + jnp.sum(ls(-q2_ref[...]))
                         + jnp.sum(ls(-q3_ref[...])) + jnp.sum(ls(-q4_ref[...]))))
        total = loss - WEIGHT_DECAY * 0.5 * jnp.sum(regp_ref[...])
        o_ref[0, 0] = -total / batch_size

    return pl.pallas_call(
        body,
        out_shape=jax.ShapeDtypeStruct((1, 1), jnp.float32),
        out_specs=pl.BlockSpec(memory_space=pltpu.SMEM),
    )(tin, tout, q1, q2, q3, q4, regp)


def kernel(input_labels, out_labels, in_embed, out_embed, edge_w, u_noise,
           v_noise, num_sampled):
    B, W1 = out_labels.shape
    W = W1 - 1
    D = in_embed.shape[1]
    NS = u_noise.shape[1]
    pairs = B * W
    ppt = pairs // NT
    nsub = ppt // SUB

    ids = input_labels[:, 1].astype(jnp.int32)
    inp_idx = jnp.tile(ids, W).reshape(NT, nsub, SUB)
    out_idx = out_labels[:, 1:].reshape(-1).astype(jnp.int32).reshape(NT, nsub, SUB)
    un_idx = u_noise.astype(jnp.int32).reshape(NT, nsub, NS * SUB)
    vn_idx = v_noise.astype(jnp.int32).reshape(NT, nsub, NS * SUB)
    idxa = jnp.concatenate([inp_idx, out_idx, un_idx], axis=2)

    tin, tout, q1, q2, q3, q4, regp = _sc_dot_kernel(B, W, NS, D, nsub)(
        in_embed, out_embed, edge_w.astype(jnp.float32), idxa, vn_idx)

    res = _tc_reduce(
        tin.reshape(pairs // 128, 128), tout.reshape(pairs // 128, 128),
        q1.reshape(pairs * NS // 128, 128), q2.reshape(pairs * NS // 128, 128),
        q3.reshape(pairs * NS // 128, 128), q4.reshape(pairs * NS // 128, 128),
        regp.reshape((NT * L) // 128, 128), float(B))
    return res[0, 0]
